# baseline (speedup 1.0000x reference)
import functools

import numpy as np
import jax
import jax.numpy as jnp
from jax import lax
from jax.experimental import pallas as pl
from jax.experimental.pallas import tpu as pltpu
from jax.experimental.pallas import tpu_sc as plsc


def _crop_consts(N, T, temporal_unit=0, seed=0):
    rng = np.random.RandomState(seed)
    crop_l = int(rng.randint(2 ** (temporal_unit + 1), T + 1))
    crop_left = int(rng.randint(T - crop_l + 1))
    crop_right = crop_left + crop_l
    crop_eleft = int(rng.randint(crop_left + 1))
    crop_eright = int(rng.randint(crop_right, T + 1))
    crop_offset = rng.randint(-crop_eleft, T - crop_eright + 1, size=N)
    starts = [int(s) for s in (crop_offset + crop_left)]
    return crop_l, starts


_CT = 432


def _lane_const(v0, v1, wid):
    lanes = lax.iota(jnp.int32, 16)
    widv = lax.broadcast_in_dim(wid, (16,), ())
    sel = lax.select(widv >= lax.broadcast_in_dim(jnp.int32(16), (16,), ()),
                     v1, v0)
    zero = lanes ^ lanes
    masked = lax.select(lanes == (widv & 15), sel, zero)
    return jnp.max(masked)


@functools.partial(jax.jit, static_argnums=(2,))
def _run(x, starts2d, crop_l):
    N, T, D = x.shape

    mesh = plsc.VectorSubcoreMesh(core_axis_name="c", subcore_axis_name="s")
    info = plsc.get_sparse_core_info()
    NC, NS = info.num_cores, info.num_subcores
    NW = NC * NS
    rows_per_w = N // NW
    n_chunks = (crop_l + _CT - 1) // _CT
    rem = crop_l - (n_chunks - 1) * _CT
    n_q = rows_per_w * n_chunks

    @functools.partial(
        pl.kernel,
        out_type=[jax.ShapeDtypeStruct((N, crop_l, D), jnp.float32),
                  jax.ShapeDtypeStruct((N, crop_l, D), jnp.float32)],
        scratch_types=[
            pltpu.VMEM((2, _CT + 8, D), jnp.float32),
            pltpu.VMEM((2 * rows_per_w, 16), jnp.int32),
            pltpu.SemaphoreType.DMA,
            pltpu.SemaphoreType.DMA,
            pltpu.SemaphoreType.DMA,
            pltpu.SemaphoreType.DMA,
        ],
        compiler_params=pltpu.CompilerParams(needs_layout_passes=False),
        mesh=mesh,
    )
    def k(x_hbm, starts_hbm, out_hbm, out2_hbm, buf_v, st_v, sem0, sem1,
          sem2, sem3):
        wid = lax.axis_index("s") * NC + lax.axis_index("c")
        sems = (sem0, sem1)
        ssems = (sem2, sem3)
        pltpu.sync_copy(starts_hbm, st_v)
        rows, bases, pads = [], [], []
        for j in range(rows_per_w):
            base = _lane_const(st_v[2 * j], st_v[2 * j + 1], wid)
            pad = base & 7
            rows.append(wid + NW * j)
            bases.append(base - pad)
            pads.append(pad)

        def chunk_info(q):
            j, c = q // n_chunks, q % n_chunks
            off = c * _CT
            L = _CT if c < n_chunks - 1 else rem
            return j, off, L

        def gather(q):
            b = q % 2
            j, off, L = chunk_info(q)
            W = ((L + 7) // 8) * 8 + 8
            return pltpu.make_async_copy(
                x_hbm.at[rows[j],
                         pl.ds(pl.multiple_of(bases[j] + off, 8), W)],
                buf_v.at[b, pl.ds(0, W)], sems[b])

        def scatters(q):
            b = q % 2
            j, off, L = chunk_info(q)
            return [pltpu.make_async_copy(
                        buf_v.at[b, pl.ds(pads[j], L)],
                        o.at[rows[j], pl.ds(off, L)], ssems[b])
                    for o in (out_hbm, out2_hbm)]

        gather(0).start()
        for q in range(n_q):
            if q >= 1:
                for cp in scatters(q - 1):
                    cp.wait()
            if q + 1 < n_q:
                gather(q + 1).start()
            gather(q).wait()
            for cp in scatters(q):
                cp.start()
        for cp in scatters(n_q - 1):
            cp.wait()

    return k(x, starts2d)


def kernel(x):
    N, T, D = x.shape
    crop_l, starts = _crop_consts(N, T)
    starts2d = jnp.asarray(np.array(starts, dtype=np.int32).reshape(-1, 16))
    out1, out2 = _run(x, starts2d, crop_l)
    return (out1, out2)


# speedup vs baseline: 2.6102x; 2.6102x over previous
import functools

import numpy as np
import jax
import jax.numpy as jnp
from jax import lax
from jax.experimental import pallas as pl
from jax.experimental.pallas import tpu as pltpu
from jax.experimental.pallas import tpu_sc as plsc


def _crop_consts(N, T, temporal_unit=0, seed=0):
    rng = np.random.RandomState(seed)
    crop_l = int(rng.randint(2 ** (temporal_unit + 1), T + 1))
    crop_left = int(rng.randint(T - crop_l + 1))
    crop_right = crop_left + crop_l
    crop_eleft = int(rng.randint(crop_left + 1))
    crop_eright = int(rng.randint(crop_right, T + 1))
    crop_offset = rng.randint(-crop_eleft, T - crop_eright + 1, size=N)
    starts = (crop_offset + crop_left).astype(np.int64)
    return crop_l, starts


@functools.partial(jax.jit, static_argnums=(2, 3))
def _run(x2d, idx, crop_l, N):
    D = 128
    B = 2 * N
    n_units = idx.shape[1]
    n_k = crop_l // 2

    mesh = plsc.VectorSubcoreMesh(core_axis_name="c", subcore_axis_name="s")
    info = plsc.get_sparse_core_info()
    NC, NS = info.num_cores, info.num_subcores
    NW = NC * NS
    n_full = n_k // NW
    n_extra = n_k - n_full * NW

    out_struct = jax.ShapeDtypeStruct((n_k * B, D), jnp.float32)

    @functools.partial(
        pl.kernel,
        out_type=[out_struct, out_struct],
        scratch_types=[
            pltpu.VMEM((2, B, D), jnp.float32),
            pltpu.VMEM((n_units, B), jnp.int32),
            pltpu.SemaphoreType.DMA,
            pltpu.SemaphoreType.DMA,
            pltpu.SemaphoreType.DMA,
            pltpu.SemaphoreType.DMA,
        ],
        compiler_params=pltpu.CompilerParams(needs_layout_passes=False),
        mesh=mesh,
    )
    def k(x_hbm, idx_hbm, out_hbm, out2_hbm, buf_v, idx_v, g0, g1, s0, s1):
        wid = lax.axis_index("s") * NC + lax.axis_index("c")
        gsems = (g0, g1)
        ssems = (s0, s1)
        pltpu.sync_copy(idx_hbm.at[wid], idx_v)

        def gather(m):
            b = m % 2
            return pltpu.make_async_copy(
                x_hbm.at[idx_v.at[m]], buf_v.at[b], gsems[b])

        def scatters(m):
            b = m % 2
            off = (wid + NW * m) * B
            return [pltpu.make_async_copy(
                        buf_v.at[b],
                        o.at[pl.ds(pl.multiple_of(off, B), B)],
                        ssems[b])
                    for o in (out_hbm, out2_hbm)]

        gather(0).start()
        for m in range(n_full):
            if m >= 1:
                for cp in scatters(m - 1):
                    cp.wait()
            if m + 1 < n_full:
                gather(m + 1).start()
            elif n_extra:
                @pl.when(wid < n_extra)
                def _():
                    gather(n_full).start()
            gather(m).wait()
            for cp in scatters(m):
                cp.start()

        if n_extra:
            @pl.when(wid < n_extra)
            def _():
                for cp in scatters(n_full - 1):
                    cp.wait()
                gather(n_full).wait()
                for cp in scatters(n_full):
                    cp.start()
                for cp in scatters(n_full):
                    cp.wait()

            @pl.when(wid >= n_extra)
            def _():
                for cp in scatters(n_full - 1):
                    cp.wait()
        else:
            for cp in scatters(n_full - 1):
                cp.wait()

    return k(x2d, idx)


def kernel(x):
    N, T, D = x.shape
    crop_l, starts = _crop_consts(N, T)
    n_k = crop_l // 2
    info = plsc.get_sparse_core_info()
    NW = info.num_cores * info.num_subcores
    n_units = (n_k + NW - 1) // NW
    i_arr = np.tile(np.arange(N, dtype=np.int64), 2)
    t_arr = np.repeat(np.arange(2, dtype=np.int64), N)
    k_ids = np.arange(n_k, dtype=np.int64)
    idx_np = ((i_arr * T + starts[i_arr]) + t_arr)[None, :] \
        + 2 * k_ids[:, None]
    pad = n_units * NW - n_k
    if pad:
        idx_np = np.concatenate([idx_np, idx_np[:pad]], axis=0)
    idx_np = idx_np.reshape(n_units, NW, 2 * N).transpose(1, 0, 2)
    idx = jnp.asarray(idx_np.astype(np.int32))
    out1, out2 = _run(x.reshape(N * T, D), idx, crop_l, N)
    shape = (n_k * 2, N, D)
    out1 = out1.reshape(shape).transpose(1, 0, 2)
    out2 = out2.reshape(shape).transpose(1, 0, 2)
    return (out1, out2)


# speedup vs baseline: 2.6613x; 1.0196x over previous
import functools

import numpy as np
import jax
import jax.numpy as jnp
from jax import lax
from jax.experimental import pallas as pl
from jax.experimental.pallas import tpu as pltpu
from jax.experimental.pallas import tpu_sc as plsc


def _crop_consts(N, T, temporal_unit=0, seed=0):
    rng = np.random.RandomState(seed)
    crop_l = int(rng.randint(2 ** (temporal_unit + 1), T + 1))
    crop_left = int(rng.randint(T - crop_l + 1))
    crop_right = crop_left + crop_l
    crop_eleft = int(rng.randint(crop_left + 1))
    crop_eright = int(rng.randint(crop_right, T + 1))
    crop_offset = rng.randint(-crop_eleft, T - crop_eright + 1, size=N)
    starts = (crop_offset + crop_left).astype(np.int64)
    return crop_l, starts


@functools.partial(jax.jit, static_argnums=(2, 3))
def _run(x2d, idx, crop_l, N):
    D = 128
    B = 2 * N
    n_units = idx.shape[1]
    n_k = crop_l // 2

    mesh = plsc.VectorSubcoreMesh(core_axis_name="c", subcore_axis_name="s")
    info = plsc.get_sparse_core_info()
    NC, NS = info.num_cores, info.num_subcores
    NW = NC * NS
    n_full = n_k // NW
    n_extra = n_k - n_full * NW

    out_struct = jax.ShapeDtypeStruct((n_k * B, D), jnp.float32)

    @functools.partial(
        pl.kernel,
        out_type=[out_struct, out_struct],
        scratch_types=[
            pltpu.VMEM((4, B, D), jnp.float32),
            pltpu.VMEM((n_units, B), jnp.int32),
            pltpu.SemaphoreType.DMA,
            pltpu.SemaphoreType.DMA,
            pltpu.SemaphoreType.DMA,
            pltpu.SemaphoreType.DMA,
            pltpu.SemaphoreType.DMA,
            pltpu.SemaphoreType.DMA,
            pltpu.SemaphoreType.DMA,
            pltpu.SemaphoreType.DMA,
        ],
        compiler_params=pltpu.CompilerParams(needs_layout_passes=False),
        mesh=mesh,
    )
    def k(x_hbm, idx_hbm, out_hbm, out2_hbm, buf_v, idx_v,
          g0, g1, g2, g3, s0, s1, s2, s3):
        wid = lax.axis_index("s") * NC + lax.axis_index("c")
        gsems = (g0, g1, g2, g3)
        ssems = (s0, s1, s2, s3)
        pltpu.sync_copy(idx_hbm.at[wid], idx_v)

        def gather(m):
            b = m % 4
            return pltpu.make_async_copy(
                x_hbm.at[idx_v.at[m]], buf_v.at[b], gsems[b])

        def scatters(m):
            b = m % 4
            off = (wid + NW * m) * B
            return [pltpu.make_async_copy(
                        buf_v.at[b],
                        o.at[pl.ds(pl.multiple_of(off, B), B)],
                        ssems[b])
                    for o in (out_hbm, out2_hbm)]

        gather(0).start()
        gather(1).start()
        for m in range(n_full):
            if m >= 2:
                for cp in scatters(m - 2):
                    cp.wait()
            if m + 2 < n_full:
                gather(m + 2).start()
            elif m + 2 == n_full and n_extra:
                @pl.when(wid < n_extra)
                def _():
                    gather(n_full).start()
            gather(m).wait()
            for cp in scatters(m):
                cp.start()

        if n_extra:
            @pl.when(wid < n_extra)
            def _():
                for cp in scatters(n_full - 2):
                    cp.wait()
                gather(n_full).wait()
                for cp in scatters(n_full):
                    cp.start()
                for cp in scatters(n_full - 1):
                    cp.wait()
                for cp in scatters(n_full):
                    cp.wait()

            @pl.when(wid >= n_extra)
            def _():
                for cp in scatters(n_full - 2):
                    cp.wait()
                for cp in scatters(n_full - 1):
                    cp.wait()
        else:
            for cp in scatters(n_full - 2):
                cp.wait()
            for cp in scatters(n_full - 1):
                cp.wait()

    return k(x2d, idx)


def kernel(x):
    N, T, D = x.shape
    crop_l, starts = _crop_consts(N, T)
    n_k = crop_l // 2
    info = plsc.get_sparse_core_info()
    NW = info.num_cores * info.num_subcores
    n_units = (n_k + NW - 1) // NW
    i_arr = np.tile(np.arange(N, dtype=np.int64), 2)
    t_arr = np.repeat(np.arange(2, dtype=np.int64), N)
    k_ids = np.arange(n_k, dtype=np.int64)
    idx_np = ((i_arr * T + starts[i_arr]) + t_arr)[None, :] \
        + 2 * k_ids[:, None]
    pad = n_units * NW - n_k
    if pad:
        idx_np = np.concatenate([idx_np, idx_np[:pad]], axis=0)
    idx_np = idx_np.reshape(n_units, NW, 2 * N).transpose(1, 0, 2)
    idx = jnp.asarray(idx_np.astype(np.int32))
    out1, out2 = _run(x.reshape(N * T, D), idx, crop_l, N)
    shape = (n_k * 2, N, D)
    out1 = out1.reshape(shape).transpose(1, 0, 2)
    out2 = out2.reshape(shape).transpose(1, 0, 2)
    return (out1, out2)
